# Initial kernel scaffold; baseline (speedup 1.0000x reference)
import jax
import jax.numpy as jnp
from jax.experimental import pallas as pl


def kernel(x, edge_index, W0, b0, W1, b1, W2, b2, Wl1, bl1, Wl2, bl2):
    raise NotImplementedError("write your pallas kernel here")


# speedup vs baseline: 10.3807x; 10.3807x over previous
import functools

import jax
import jax.numpy as jnp
from jax import lax
from jax.experimental import pallas as pl
from jax.experimental.pallas import tpu as pltpu
from jax.experimental.pallas import tpu_sc as plsc

EPS = 1e-5
CH = 128
NC = 2
NS = 16
NW = NC * NS
ZR = 1000


def _deg_kernel_body(dstp, ones_h, z_h, out, dst_v, ones_v, acc):
    cid = lax.axis_index("c")
    sid = lax.axis_index("s")
    wid = cid * NS + sid
    n_chunks = dstp.shape[1]
    nz = out.shape[1] // ZR

    pltpu.sync_copy(dstp.at[wid], dst_v)
    pltpu.sync_copy(ones_h, ones_v)

    @pl.when(sid < nz)
    def _():
        pltpu.sync_copy(z_h, acc.at[pl.ds(sid * ZR, ZR)])

    plsc.subcore_barrier()

    def body(c, carry):
        pltpu.sync_copy(ones_v, acc.at[dst_v.at[c]], add=True)
        return carry

    lax.fori_loop(0, n_chunks, body, 0)
    plsc.subcore_barrier()

    @pl.when(sid < nz)
    def _():
        pltpu.sync_copy(acc.at[pl.ds(sid * ZR, ZR)],
                        out.at[cid, pl.ds(sid * ZR, ZR)])


def _scatter_kernel_body(g_h, srcp, dstp, z_h, out, src_v, dst_v, buf, acc):
    cid = lax.axis_index("c")
    sid = lax.axis_index("s")
    wid = cid * NS + sid
    n_chunks = srcp.shape[1]
    nz = out.shape[1] // ZR

    pltpu.sync_copy(srcp.at[wid], src_v)
    pltpu.sync_copy(dstp.at[wid], dst_v)

    @pl.when(sid < nz)
    def _():
        pltpu.sync_copy(z_h, acc.at[pl.ds(sid * ZR, ZR)])

    plsc.subcore_barrier()

    def body(c, carry):
        pltpu.sync_copy(g_h.at[src_v.at[c]], buf)
        pltpu.sync_copy(buf, acc.at[dst_v.at[c]], add=True)
        return carry

    lax.fori_loop(0, n_chunks, body, 0)
    plsc.subcore_barrier()

    @pl.when(sid < nz)
    def _():
        pltpu.sync_copy(acc.at[pl.ds(sid * ZR, ZR)],
                        out.at[cid, pl.ds(sid * ZR, ZR)])


def _dinv(degp_ref):
    deg = degp_ref[0, :, 0:1] + degp_ref[1, :, 0:1] + 1.0
    return lax.rsqrt(deg)


def _tc_first(degp_ref, x_ref, w_ref, g_ref):
    dinv = _dinv(degp_ref)
    h = jnp.dot(x_ref[...], w_ref[...], preferred_element_type=jnp.float32)
    g_ref[...] = h * dinv


def _bn_relu(pre):
    m = jnp.mean(pre, axis=0, keepdims=True)
    c = pre - m
    v = jnp.mean(c * c, axis=0, keepdims=True)
    return jnp.maximum(c * lax.rsqrt(v + EPS), 0.0)


def _tc_mid(p_ref, g_ref, degp_ref, b_ref, w_ref, gout_ref):
    dinv = _dinv(degp_ref)
    agg = p_ref[0] + p_ref[1] + g_ref[...]
    pre = agg * dinv + b_ref[...]
    y = _bn_relu(pre)
    h = jnp.dot(y, w_ref[...], preferred_element_type=jnp.float32)
    gout_ref[...] = h * dinv


def _tc_head(p_ref, g_ref, degp_ref, b_ref, wl1_ref, bl1_ref, wl2_ref,
             bl2_ref, o_ref):
    dinv = _dinv(degp_ref)
    agg = p_ref[0] + p_ref[1] + g_ref[...]
    pre = agg * dinv + b_ref[...]
    y = _bn_relu(pre)
    t = jnp.dot(y, wl1_ref[...], preferred_element_type=jnp.float32)
    t = _bn_relu(t + bl1_ref[...])
    o_ref[...] = jnp.dot(t, wl2_ref[...],
                         preferred_element_type=jnp.float32) + bl2_ref[...]


def kernel(x, edge_index, W0, b0, W1, b1, W2, b2, Wl1, bl1, Wl2, bl2):
    n, d = x.shape
    h = W0.shape[1]
    e = edge_index.shape[1]
    assert n % ZR == 0 and n // ZR <= NS
    n_chunks = -(-e // (NW * CH))
    pad_e = NW * n_chunks * CH - e

    src = edge_index[0]
    dst = edge_index[1]
    srcp = jnp.concatenate(
        [src, jnp.zeros((pad_e,), src.dtype)]).reshape(NW, n_chunks, CH)
    dstp = jnp.concatenate(
        [dst, jnp.full((pad_e,), n, dst.dtype)]).reshape(NW, n_chunks, CH)

    ones128 = jnp.ones((CH, h), jnp.float32)
    z128 = jnp.zeros((ZR, h), jnp.float32)

    mesh = plsc.VectorSubcoreMesh(core_axis_name="c", subcore_axis_name="s")

    deg_call = functools.partial(
        pl.kernel, _deg_kernel_body,
        out_type=jax.ShapeDtypeStruct((NC, n, h), jnp.float32),
        mesh=mesh,
        scratch_types=[
            pltpu.VMEM((n_chunks, CH), jnp.int32),
            pltpu.VMEM((CH, h), jnp.float32),
            pltpu.VMEM_SHARED((n + 1, h), jnp.float32),
        ],
    )()
    degp = deg_call(dstp, ones128, z128)

    scatter_call = functools.partial(
        pl.kernel, _scatter_kernel_body,
        out_type=jax.ShapeDtypeStruct((NC, n, h), jnp.float32),
        mesh=mesh,
        scratch_types=[
            pltpu.VMEM((n_chunks, CH), jnp.int32),
            pltpu.VMEM((n_chunks, CH), jnp.int32),
            pltpu.VMEM((CH, h), jnp.float32),
            pltpu.VMEM_SHARED((n + 1, h), jnp.float32),
        ],
    )()

    b0r = b0.reshape(1, h)
    b1r = b1.reshape(1, h)
    b2r = b2.reshape(1, h)
    bl1r = bl1.reshape(1, h)
    wl2p = jnp.pad(Wl2, ((0, 0), (0, 8 - Wl2.shape[1])))
    bl2p = jnp.pad(bl2, (0, 8 - bl2.shape[0])).reshape(1, 8)

    g0 = pl.pallas_call(
        _tc_first,
        out_shape=jax.ShapeDtypeStruct((n, h), jnp.float32),
    )(degp, x, W0)

    p0 = scatter_call(g0, srcp, dstp, z128)

    g1 = pl.pallas_call(
        _tc_mid,
        out_shape=jax.ShapeDtypeStruct((n, h), jnp.float32),
    )(p0, g0, degp, b0r, W1)

    p1 = scatter_call(g1, srcp, dstp, z128)

    g2 = pl.pallas_call(
        _tc_mid,
        out_shape=jax.ShapeDtypeStruct((n, h), jnp.float32),
    )(p1, g1, degp, b1r, W2)

    p2 = scatter_call(g2, srcp, dstp, z128)

    out8 = pl.pallas_call(
        _tc_head,
        out_shape=jax.ShapeDtypeStruct((n, 8), jnp.float32),
    )(p2, g2, degp, b2r, Wl1, bl1r, wl2p, bl2p)

    return out8[:, :Wl2.shape[1]]
